# baseline (speedup 1.0000x reference)
import jax
import jax.numpy as jnp
from jax.experimental import pallas as pl
from jax.experimental.pallas import tpu as pltpu

_OUT_D = 8192
_IN_D = 2048
_B = 128
_K = 164
_BOOST = 100.0
_JBLK = 512
_NJ = _OUT_D // _JBLK


def _count_ge(u, thr):
    return jnp.sum((u >= thr).astype(jnp.int32), axis=1, keepdims=True)


def _pooler_kernel(x_ref, conn_ref, avg_ref, out_ref, lo_ref, max_ref):
    j = pl.program_id(0)
    avg = avg_ref[...]
    s = jnp.sum(avg)
    avg_blk = avg_ref[:, pl.ds(j * _JBLK, _JBLK)]
    neigh = (s - avg_blk) / (_OUT_D - 1)
    boost = jnp.exp(-_BOOST * (avg_blk - neigh))
    conn_b = conn_ref[...].astype(jnp.bfloat16)
    ov = jnp.dot(x_ref[...], conn_b, preferred_element_type=jnp.float32)
    ovb = ov * boost
    out_ref[:, pl.ds(j * _JBLK, _JBLK)] = ovb

    ub = jax.lax.bitcast_convert_type(ovb, jnp.int32)
    bmax = jnp.max(ub, axis=1, keepdims=True)

    @pl.when(j == 0)
    def _seed():
        max_ref[...] = bmax

        def vbody(_, carry):
            lo, hi = carry
            mid = lo + jax.lax.div(hi - lo, 2)
            ge = _count_ge(ub, mid) >= _K
            return jnp.where(ge, mid, lo), jnp.where(ge, hi, mid)

        lo0 = jnp.zeros((_B, 1), jnp.int32)
        hi0 = bmax + 1
        t0, _ = jax.lax.fori_loop(0, 31, vbody, (lo0, hi0))
        lo_ref[...] = t0

    @pl.when(j > 0)
    def _accum_max():
        max_ref[...] = jnp.maximum(max_ref[...], bmax)

    @pl.when(j == _NJ - 1)
    def _select():
        u = jax.lax.bitcast_convert_type(out_ref[...], jnp.int32)

        def vcond(carry):
            lo, hi = carry
            return jnp.any(hi - lo > 1)

        def vbody(carry):
            lo, hi = carry
            mid = lo + jax.lax.div(hi - lo, 2)
            ge = _count_ge(u, mid) >= _K
            return jnp.where(ge, mid, lo), jnp.where(ge, hi, mid)

        t, _ = jax.lax.while_loop(vcond, vbody, (lo_ref[...], max_ref[...] + 1))

        gt = u > t
        c = jnp.sum(gt.astype(jnp.int32), axis=1, keepdims=True)
        m = _K - c
        eq = u == t
        idx = jax.lax.broadcasted_iota(jnp.int32, (_B, _OUT_D), 1)

        def tcond(carry):
            need, _ = carry
            return jnp.max(need) > 0

        def tbody(carry):
            need, last = carry
            avail = eq & (idx > last)
            fi = jnp.min(jnp.where(avail, idx, _OUT_D), axis=1, keepdims=True)
            act = need > 0
            return need - act.astype(jnp.int32), jnp.where(act, fi, last)

        _, last = jax.lax.while_loop(
            tcond, tbody, (m, jnp.full((_B, 1), -1, jnp.int32)))

        out_ref[...] = (gt | (eq & (idx <= last))).astype(jnp.float32)


def kernel(x, connection, avg_activation):
    return pl.pallas_call(
        _pooler_kernel,
        grid=(_NJ,),
        in_specs=[
            pl.BlockSpec((_B, _IN_D), lambda j: (0, 0)),
            pl.BlockSpec((_IN_D, _JBLK), lambda j: (0, j)),
            pl.BlockSpec((1, _OUT_D), lambda j: (0, 0)),
        ],
        out_specs=pl.BlockSpec((_B, _OUT_D), lambda j: (0, 0)),
        out_shape=jax.ShapeDtypeStruct((_B, _OUT_D), jnp.float32),
        scratch_shapes=[
            pltpu.VMEM((_B, 1), jnp.int32),
            pltpu.VMEM((_B, 1), jnp.int32),
        ],
    )(x, connection, avg_activation)


# speedup vs baseline: 1.2056x; 1.2056x over previous
import jax
import jax.numpy as jnp
from jax.experimental import pallas as pl
from jax.experimental.pallas import tpu as pltpu

_OUT_D = 8192
_IN_D = 2048
_B = 128
_K = 164
_BOOST = 100.0
_JBLK = 1024
_NJ = _OUT_D // _JBLK


def _count_ge(u, thr):
    return jnp.sum((u >= thr).astype(jnp.int32), axis=1, keepdims=True)


def _sum16(x):
    w = x.shape[1]
    while w > 128:
        x = x[:, : w // 2] + x[:, w // 2 :]
        w //= 2
    return jnp.sum(x.astype(jnp.int32), axis=1, keepdims=True)


def _pooler_kernel(x_ref, conn_ref, avg_ref, out_ref, lo_ref, max_ref):
    j = pl.program_id(0)
    avg = avg_ref[...]
    s = jnp.sum(avg)
    avg_blk = avg_ref[:, pl.ds(j * _JBLK, _JBLK)]
    neigh = (s - avg_blk) / (_OUT_D - 1)
    boost = jnp.exp(-_BOOST * (avg_blk - neigh))
    ov = jnp.dot(x_ref[...], conn_ref[...], preferred_element_type=jnp.float32)
    ovb = ov * boost
    out_ref[:, pl.ds(j * _JBLK, _JBLK)] = ovb

    ub = jax.lax.bitcast_convert_type(ovb, jnp.int32)
    bmax = jnp.max(ub, axis=1, keepdims=True)

    @pl.when(j == 0)
    def _seed():
        max_ref[...] = bmax

        def vbody(_, carry):
            lo, hi = carry
            mid = lo + jax.lax.div(hi - lo, 2)
            ge = _count_ge(ub, mid) >= _K
            return jnp.where(ge, mid, lo), jnp.where(ge, hi, mid)

        lo0 = jnp.zeros((_B, 1), jnp.int32)
        hi0 = bmax + 1
        t0, _ = jax.lax.fori_loop(0, 31, vbody, (lo0, hi0))
        lo_ref[...] = t0

    @pl.when(j > 0)
    def _accum_max():
        max_ref[...] = jnp.maximum(max_ref[...], bmax)

    @pl.when(j == _NJ - 1)
    def _select():
        u = jax.lax.bitcast_convert_type(out_ref[...], jnp.int32)

        h16 = (u >> 16).astype(jnp.int16)

        def acond(carry):
            lo, hi, _ = carry
            return jnp.any(hi - lo > 1)

        def abody(carry):
            lo, hi, ch = carry
            mid = lo + jax.lax.div(hi - lo, 2)
            cnt = _sum16((h16 >= mid.astype(jnp.int16)).astype(jnp.int16))
            ge = cnt >= _K
            return (jnp.where(ge, mid, lo), jnp.where(ge, hi, mid),
                    jnp.where(ge, ch, cnt))

        t16, _, cnt_a = jax.lax.while_loop(
            acond, abody,
            (lo_ref[...] >> 16, (max_ref[...] >> 16) + 1,
             jnp.zeros((_B, 1), jnp.int32)))
        k2 = _K - cnt_a

        l15 = ((u >> 1) & 0x7FFF).astype(jnp.int16)
        eqv = jnp.where(h16 == t16.astype(jnp.int16), l15, jnp.int16(-1))

        def bcond(carry):
            lo, hi, _ = carry
            return jnp.any(hi - lo > 1)

        def bbody(carry):
            lo, hi, ch = carry
            mid = lo + jax.lax.div(hi - lo, 2)
            cnt = _sum16((eqv >= mid.astype(jnp.int16)).astype(jnp.int16))
            ge = cnt >= k2
            return (jnp.where(ge, mid, lo), jnp.where(ge, hi, mid),
                    jnp.where(ge, ch, cnt))

        t15, _, cnt_b = jax.lax.while_loop(
            bcond, bbody,
            (jnp.zeros((_B, 1), jnp.int32),
             jnp.full((_B, 1), 1 << 15, jnp.int32),
             jnp.zeros((_B, 1), jnp.int32)))

        base = (t16 << 16) | (t15 << 1)
        c1 = _count_ge(u, base + 1)
        up = c1 >= _K
        t = jnp.where(up, base + 1, base)
        c = jnp.where(up, cnt_a + cnt_b, c1)
        m = _K - c

        gt = u > t
        eq = u == t
        idx = jax.lax.broadcasted_iota(jnp.int32, (_B, _OUT_D), 1)

        def tcond(carry):
            need, _ = carry
            return jnp.max(need) > 0

        def tbody(carry):
            need, last = carry
            avail = eq & (idx > last)
            fi = jnp.min(jnp.where(avail, idx, _OUT_D), axis=1, keepdims=True)
            act = need > 0
            return need - act.astype(jnp.int32), jnp.where(act, fi, last)

        _, last = jax.lax.while_loop(
            tcond, tbody, (m, jnp.full((_B, 1), -1, jnp.int32)))

        out_ref[...] = (gt | (eq & (idx <= last))).astype(jnp.float32)


def kernel(x, connection, avg_activation):
    return pl.pallas_call(
        _pooler_kernel,
        grid=(_NJ,),
        in_specs=[
            pl.BlockSpec((_B, _IN_D), lambda j: (0, 0)),
            pl.BlockSpec((_IN_D, _JBLK), lambda j: (0, j)),
            pl.BlockSpec((1, _OUT_D), lambda j: (0, 0)),
        ],
        out_specs=pl.BlockSpec((_B, _OUT_D), lambda j: (0, 0)),
        out_shape=jax.ShapeDtypeStruct((_B, _OUT_D), jnp.float32),
        scratch_shapes=[
            pltpu.VMEM((_B, 1), jnp.int32),
            pltpu.VMEM((_B, 1), jnp.int32),
        ],
    )(x, connection, avg_activation)


# speedup vs baseline: 1.2981x; 1.0767x over previous
import jax
import jax.numpy as jnp
from jax.experimental import pallas as pl
from jax.experimental.pallas import tpu as pltpu

_OUT_D = 8192
_IN_D = 2048
_B = 128
_K = 164
_BOOST = 100.0
_JBLK = 1024
_NJ = _OUT_D // _JBLK


def _count_ge(u, thr):
    return jnp.sum((u >= thr).astype(jnp.int32), axis=1, keepdims=True)


def _sum16(x):
    w = x.shape[1]
    while w > 128:
        x = x[:, : w // 2] + x[:, w // 2 :]
        w //= 2
    return jnp.sum(x.astype(jnp.int32), axis=1, keepdims=True)


def _pooler_kernel(x_ref, conn_ref, avg_ref, out_ref, lo_ref, max_ref):
    j = pl.program_id(0)
    avg = avg_ref[...]
    s = jnp.sum(avg)
    avg_blk = avg_ref[:, pl.ds(j * _JBLK, _JBLK)]
    neigh = (s - avg_blk) / (_OUT_D - 1)
    boost = jnp.exp(-_BOOST * (avg_blk - neigh))
    ov = jnp.dot(x_ref[...], conn_ref[...], preferred_element_type=jnp.float32)
    ovb = ov * boost
    out_ref[:, pl.ds(j * _JBLK, _JBLK)] = ovb

    ub = jax.lax.bitcast_convert_type(ovb, jnp.int32)
    bmax = jnp.max(ub, axis=1, keepdims=True)

    @pl.when(j == 0)
    def _seed():
        max_ref[...] = bmax

        ubh = (ub >> 16).astype(jnp.int16)

        def vbody(_, carry):
            lo, hi = carry
            mid = lo + jax.lax.div(hi - lo, 2)
            cnt = _sum16((ubh >= mid.astype(jnp.int16)).astype(jnp.int16))
            ge = cnt >= _K
            return jnp.where(ge, mid, lo), jnp.where(ge, hi, mid)

        lo0 = jnp.zeros((_B, 1), jnp.int32)
        hi0 = (bmax >> 16) + 1
        t0, _ = jax.lax.fori_loop(0, 15, vbody, (lo0, hi0))
        lo_ref[...] = t0 << 16

    @pl.when(j > 0)
    def _accum_max():
        max_ref[...] = jnp.maximum(max_ref[...], bmax)

    @pl.when(j == _NJ - 1)
    def _select():
        u = jax.lax.bitcast_convert_type(out_ref[...], jnp.int32)

        h16 = (u >> 16).astype(jnp.int16)

        def acond(carry):
            lo, hi, _ = carry
            return jnp.any(hi - lo > 1)

        def abody(carry):
            lo, hi, ch = carry
            mid = lo + jax.lax.div(hi - lo, 2)
            cnt = _sum16((h16 >= mid.astype(jnp.int16)).astype(jnp.int16))
            ge = cnt >= _K
            return (jnp.where(ge, mid, lo), jnp.where(ge, hi, mid),
                    jnp.where(ge, ch, cnt))

        t16, _, cnt_a = jax.lax.while_loop(
            acond, abody,
            (lo_ref[...] >> 16, (max_ref[...] >> 16) + 1,
             jnp.zeros((_B, 1), jnp.int32)))
        k2 = _K - cnt_a

        l15 = ((u >> 1) & 0x7FFF).astype(jnp.int16)
        eqv = jnp.where(h16 == t16.astype(jnp.int16), l15, jnp.int16(-1))

        def bcond(carry):
            lo, hi, _ = carry
            return jnp.any(hi - lo > 1)

        def bbody(carry):
            lo, hi, ch = carry
            mid = lo + jax.lax.div(hi - lo, 2)
            cnt = _sum16((eqv >= mid.astype(jnp.int16)).astype(jnp.int16))
            ge = cnt >= k2
            return (jnp.where(ge, mid, lo), jnp.where(ge, hi, mid),
                    jnp.where(ge, ch, cnt))

        t15, _, cnt_b = jax.lax.while_loop(
            bcond, bbody,
            (jnp.zeros((_B, 1), jnp.int32),
             jnp.full((_B, 1), 1 << 15, jnp.int32),
             jnp.zeros((_B, 1), jnp.int32)))

        base = (t16 << 16) | (t15 << 1)
        c1 = _count_ge(u, base + 1)
        up = c1 >= _K
        t = jnp.where(up, base + 1, base)
        c = jnp.where(up, cnt_a + cnt_b, c1)
        m = _K - c

        gt = u > t
        eq = u == t
        idx = jax.lax.broadcasted_iota(jnp.int32, (_B, _OUT_D), 1)

        def tcond(carry):
            need, _ = carry
            return jnp.max(need) > 0

        def tbody(carry):
            need, last = carry
            avail = eq & (idx > last)
            fi = jnp.min(jnp.where(avail, idx, _OUT_D), axis=1, keepdims=True)
            act = need > 0
            return need - act.astype(jnp.int32), jnp.where(act, fi, last)

        _, last = jax.lax.while_loop(
            tcond, tbody, (m, jnp.full((_B, 1), -1, jnp.int32)))

        out_ref[...] = (gt | (eq & (idx <= last))).astype(jnp.float32)


def kernel(x, connection, avg_activation):
    return pl.pallas_call(
        _pooler_kernel,
        grid=(_NJ,),
        in_specs=[
            pl.BlockSpec((_B, _IN_D), lambda j: (0, 0)),
            pl.BlockSpec((_IN_D, _JBLK), lambda j: (0, j)),
            pl.BlockSpec((1, _OUT_D), lambda j: (0, 0)),
        ],
        out_specs=pl.BlockSpec((_B, _OUT_D), lambda j: (0, 0)),
        out_shape=jax.ShapeDtypeStruct((_B, _OUT_D), jnp.float32),
        scratch_shapes=[
            pltpu.VMEM((_B, 1), jnp.int32),
            pltpu.VMEM((_B, 1), jnp.int32),
        ],
    )(x, connection, avg_activation)


# speedup vs baseline: 1.3099x; 1.0091x over previous
import jax
import jax.numpy as jnp
from jax.experimental import pallas as pl
from jax.experimental.pallas import tpu as pltpu

_OUT_D = 8192
_IN_D = 2048
_B = 128
_K = 164
_BOOST = 100.0
_JBLK = 1024
_NJ = _OUT_D // _JBLK


def _count_ge(u, thr):
    return jnp.sum((u >= thr).astype(jnp.int32), axis=1, keepdims=True)


def _sum16(x):
    w = x.shape[1]
    while w > 128:
        x = x[:, : w // 2] + x[:, w // 2 :]
        w //= 2
    return jnp.sum(x.astype(jnp.int32), axis=1, keepdims=True)


def _pooler_kernel(x_ref, conn_ref, avg_ref, out_ref, lo_ref, max_ref, h16_ref, l15_ref):
    j = pl.program_id(0)
    avg = avg_ref[...]
    s = jnp.sum(avg)
    avg_blk = avg_ref[:, pl.ds(j * _JBLK, _JBLK)]
    neigh = (s - avg_blk) / (_OUT_D - 1)
    boost = jnp.exp(-_BOOST * (avg_blk - neigh))
    ov = jnp.dot(x_ref[...], conn_ref[...], preferred_element_type=jnp.float32)
    ovb = ov * boost
    out_ref[:, pl.ds(j * _JBLK, _JBLK)] = ovb

    ub = jax.lax.bitcast_convert_type(ovb, jnp.int32)
    bmax = jnp.max(ub, axis=1, keepdims=True)
    h16_ref[:, pl.ds(j * _JBLK, _JBLK)] = (ub >> 16).astype(jnp.int16)
    l15_ref[:, pl.ds(j * _JBLK, _JBLK)] = ((ub >> 1) & 0x7FFF).astype(jnp.int16)

    @pl.when(j == 0)
    def _seed():
        max_ref[...] = bmax

        ubh = (ub >> 16).astype(jnp.int16)

        def vbody(_, carry):
            lo, hi = carry
            mid = lo + jax.lax.div(hi - lo, 2)
            cnt = _sum16((ubh >= mid.astype(jnp.int16)).astype(jnp.int16))
            ge = cnt >= _K
            return jnp.where(ge, mid, lo), jnp.where(ge, hi, mid)

        lo0 = jnp.zeros((_B, 1), jnp.int32)
        hi0 = (bmax >> 16) + 1
        t0, _ = jax.lax.fori_loop(0, 15, vbody, (lo0, hi0))
        lo_ref[...] = t0 << 16

    @pl.when(j > 0)
    def _accum_max():
        max_ref[...] = jnp.maximum(max_ref[...], bmax)

    @pl.when(j == _NJ - 1)
    def _select():
        u = jax.lax.bitcast_convert_type(out_ref[...], jnp.int32)

        h16 = h16_ref[...]

        def acond(carry):
            lo, hi, _ = carry
            return jnp.any(hi - lo > 1)

        def abody(carry):
            lo, hi, ch = carry
            mid = lo + jax.lax.div(hi - lo, 2)
            cnt = _sum16((h16 >= mid.astype(jnp.int16)).astype(jnp.int16))
            ge = cnt >= _K
            return (jnp.where(ge, mid, lo), jnp.where(ge, hi, mid),
                    jnp.where(ge, ch, cnt))

        t16, _, cnt_a = jax.lax.while_loop(
            acond, abody,
            (lo_ref[...] >> 16, (max_ref[...] >> 16) + 1,
             jnp.zeros((_B, 1), jnp.int32)))
        k2 = _K - cnt_a

        eqv = jnp.where(h16 == t16.astype(jnp.int16), l15_ref[...],
                        jnp.int16(-1))

        def bcond(carry):
            lo, hi, _ = carry
            return jnp.any(hi - lo > 1)

        def bbody(carry):
            lo, hi, ch = carry
            mid = lo + jax.lax.div(hi - lo, 2)
            cnt = _sum16((eqv >= mid.astype(jnp.int16)).astype(jnp.int16))
            ge = cnt >= k2
            return (jnp.where(ge, mid, lo), jnp.where(ge, hi, mid),
                    jnp.where(ge, ch, cnt))

        t15, _, cnt_b = jax.lax.while_loop(
            bcond, bbody,
            (jnp.zeros((_B, 1), jnp.int32),
             jnp.full((_B, 1), 1 << 15, jnp.int32),
             jnp.zeros((_B, 1), jnp.int32)))

        base = (t16 << 16) | (t15 << 1)
        c1 = _count_ge(u, base + 1)
        up = c1 >= _K
        t = jnp.where(up, base + 1, base)
        c = jnp.where(up, cnt_a + cnt_b, c1)
        m = _K - c

        gt = u > t
        eq = u == t
        idx = jax.lax.broadcasted_iota(jnp.int32, (_B, _OUT_D), 1)

        def tcond(carry):
            need, _ = carry
            return jnp.max(need) > 0

        def tbody(carry):
            need, last = carry
            avail = eq & (idx > last)
            fi = jnp.min(jnp.where(avail, idx, _OUT_D), axis=1, keepdims=True)
            act = need > 0
            return need - act.astype(jnp.int32), jnp.where(act, fi, last)

        _, last = jax.lax.while_loop(
            tcond, tbody, (m, jnp.full((_B, 1), -1, jnp.int32)))

        out_ref[...] = (gt | (eq & (idx <= last))).astype(jnp.float32)


def kernel(x, connection, avg_activation):
    return pl.pallas_call(
        _pooler_kernel,
        grid=(_NJ,),
        in_specs=[
            pl.BlockSpec((_B, _IN_D), lambda j: (0, 0)),
            pl.BlockSpec((_IN_D, _JBLK), lambda j: (0, j)),
            pl.BlockSpec((1, _OUT_D), lambda j: (0, 0)),
        ],
        out_specs=pl.BlockSpec((_B, _OUT_D), lambda j: (0, 0)),
        out_shape=jax.ShapeDtypeStruct((_B, _OUT_D), jnp.float32),
        scratch_shapes=[
            pltpu.VMEM((_B, 1), jnp.int32),
            pltpu.VMEM((_B, 1), jnp.int32),
            pltpu.VMEM((_B, _OUT_D), jnp.int16),
            pltpu.VMEM((_B, _OUT_D), jnp.int16),
        ],
    )(x, connection, avg_activation)
